# baseline (speedup 1.0000x reference)
import functools

import jax
import jax.numpy as jnp
from jax import lax
from jax.experimental import pallas as pl
from jax.experimental.pallas import tpu as pltpu
from jax.experimental.pallas import tpu_sc as plsc

N = 10000
D = 128
E = 320000

NC = 2
NS = 16
NW = NC * NS
LANES = 16

CHUNK = 128
NK = 78
EPW = NK * CHUNK
NTAIL = (E - NW * EPW) // CHUNK
GROUPS = CHUNK // LANES
NBUF = 3

ROWS_PER_TILE = 624
ROWS_TAIL = N - NS * ROWS_PER_TILE

_mesh = plsc.VectorSubcoreMesh(core_axis_name="c", subcore_axis_name="s")


@functools.partial(
    pl.kernel,
    out_type=jax.ShapeDtypeStruct((NC, N, D), jnp.float32),
    mesh=_mesh,
    scratch_types=[
        [pltpu.VMEM((CHUNK,), jnp.int32) for _ in range(NBUF)],
        [pltpu.VMEM((CHUNK,), jnp.int32) for _ in range(NBUF)],
        [pltpu.VMEM((CHUNK,), jnp.float32) for _ in range(NBUF)],
        [pltpu.VMEM((CHUNK, D), jnp.float32) for _ in range(NBUF)],
        pltpu.VMEM_SHARED((N, D), jnp.float32),
        [pltpu.SemaphoreType.DMA for _ in range(NBUF)],
        [pltpu.SemaphoreType.DMA for _ in range(NBUF)],
        [pltpu.SemaphoreType.DMA for _ in range(NBUF)],
        [pltpu.SemaphoreType.DMA for _ in range(NBUF)],
        [pltpu.SemaphoreType.DMA for _ in range(NBUF)],
    ],
)
def _scatter_add_sc(x_hbm, ei_hbm, ev_hbm, out_hbm,
                    srcb, dstb, evb, rows, acc_sh,
                    srcsem, dstsem, evsem, gsem, scsem):
    c = lax.axis_index("c")
    s = lax.axis_index("s")
    wid = s * NC + c
    ebase = wid * EPW

    def zero_body(r, carry):
        for j in range(D // LANES):
            rows[0][r, pl.ds(j * LANES, LANES)] = jnp.zeros(
                (LANES,), jnp.float32)
        return carry

    lax.fori_loop(0, CHUNK, zero_body, 0)
    row0 = s * ROWS_PER_TILE
    for i in range(ROWS_PER_TILE // CHUNK):
        pltpu.sync_copy(rows[0], acc_sh.at[pl.ds(row0 + i * CHUNK, CHUNK)])
    rem = ROWS_PER_TILE % CHUNK
    pltpu.sync_copy(rows[0].at[pl.ds(0, rem)],
                    acc_sh.at[pl.ds(row0 + ROWS_PER_TILE - rem, rem)])

    @pl.when(s == 0)
    def _zero_tail():
        pltpu.sync_copy(rows[0].at[pl.ds(0, ROWS_TAIL)],
                        acc_sh.at[pl.ds(NS * ROWS_PER_TILE, ROWS_TAIL)])

    plsc.subcore_barrier()

    def start_src(z, base):
        pltpu.async_copy(ei_hbm.at[pl.ds(E + base, CHUNK)], srcb[z], srcsem[z])

    def start_dst(z, base):
        pltpu.async_copy(ei_hbm.at[pl.ds(base, CHUNK)], dstb[z], dstsem[z])

    def start_ev(z, base):
        pltpu.async_copy(ev_hbm.at[pl.ds(base, CHUNK)], evb[z], evsem[z])

    def wait_1d(hbm, buf, sem):
        pltpu.make_async_copy(hbm.at[pl.ds(0, CHUNK)], buf, sem).wait()

    def start_gather(z):
        pltpu.async_copy(x_hbm.at[srcb[z]], rows[z], gsem[z])

    def wait_gather(z):
        pltpu.make_async_copy(x_hbm.at[srcb[z]], rows[z], gsem[z]).wait()

    def start_scatter(z):
        pltpu.async_copy(rows[z], acc_sh.at[dstb[z]], scsem[z], add=True)

    def wait_scatter(z):
        pltpu.make_async_copy(rows[z], acc_sh.at[dstb[z]], scsem[z]).wait()

    def scale(rows_b, ev_b):
        def group_body(g, carry):
            ev16 = ev_b[pl.ds(g * LANES, LANES)]
            for i in range(LANES):
                evs = jnp.full((LANES,), ev16[i], jnp.float32)
                e = g * LANES + i
                for j in range(D // LANES):
                    sl = pl.ds(j * LANES, LANES)
                    rows_b[e, sl] = rows_b[e, sl] * evs
            return carry

        lax.fori_loop(0, GROUPS, group_body, 0)

    for kk in range(NBUF):
        start_src(kk, ebase + kk * CHUNK)
        start_ev(kk, ebase + kk * CHUNK)
    for kk in range(2):
        start_dst(kk, ebase + kk * CHUNK)
    for kk in range(2):
        wait_1d(ei_hbm, srcb[kk], srcsem[kk])
        start_gather(kk)

    def ring_body(q, carry):
        for u in range(NBUF):
            k = NBUF * q + u
            z, zp = u, (u - 1) % NBUF
            wait_gather(z)
            wait_1d(ev_hbm, evb[z], evsem[z])
            scale(rows[z], evb[z])
            @pl.when(k > 0)
            def _retire():
                wait_scatter(zp)

            @pl.when(k + 2 < NK)
            def _dst_next():
                start_dst(zp, ebase + (k + 2) * CHUNK)

            @pl.when(k + NBUF < NK)
            def _srcev_next():
                start_src(z, ebase + (k + NBUF) * CHUNK)
                start_ev(z, ebase + (k + NBUF) * CHUNK)

            @pl.when(k + 2 < NK)
            def _gather_next():
                wait_1d(ei_hbm, srcb[zp], srcsem[zp])
                start_gather(zp)

            wait_1d(ei_hbm, dstb[z], dstsem[z])
            start_scatter(z)
        return carry

    lax.fori_loop(0, NK // NBUF, ring_body, 0)
    wait_scatter((NK - 1) % NBUF)

    @pl.when(wid < NTAIL)
    def _tail():
        tbase = NW * EPW + wid * CHUNK
        pltpu.sync_copy(ei_hbm.at[pl.ds(E + tbase, CHUNK)], srcb[0])
        pltpu.sync_copy(ei_hbm.at[pl.ds(tbase, CHUNK)], dstb[0])
        pltpu.sync_copy(ev_hbm.at[pl.ds(tbase, CHUNK)], evb[0])
        pltpu.async_copy(x_hbm.at[srcb[0]], rows[0], gsem[0]).wait()
        scale(rows[0], evb[0])
        pltpu.sync_copy(rows[0], acc_sh.at[dstb[0]], add=True)

    plsc.subcore_barrier()
    pltpu.sync_copy(acc_sh.at[pl.ds(row0, ROWS_PER_TILE)],
                    out_hbm.at[c, pl.ds(row0, ROWS_PER_TILE)])

    @pl.when(s == 0)
    def _drain_tail():
        pltpu.sync_copy(acc_sh.at[pl.ds(NS * ROWS_PER_TILE, ROWS_TAIL)],
                        out_hbm.at[c, pl.ds(NS * ROWS_PER_TILE, ROWS_TAIL)])


BLK = 1000


def _combine_mm_body(p0_ref, p1_ref, w_ref, out_ref):
    a = p0_ref[0] + p1_ref[0]
    out_ref[...] = lax.dot_general(
        a, w_ref[...], (((1,), (1,)), ((), ())),
        preferred_element_type=jnp.float32)


def _combine_matmul(partials, W):
    return pl.pallas_call(
        _combine_mm_body,
        grid=(N // BLK,),
        in_specs=[
            pl.BlockSpec((1, BLK, D), lambda i: (0, i, 0)),
            pl.BlockSpec((1, BLK, D), lambda i: (1, i, 0)),
            pl.BlockSpec((D, D), lambda i: (0, 0)),
        ],
        out_specs=pl.BlockSpec((BLK, D), lambda i: (i, 0)),
        out_shape=jax.ShapeDtypeStruct((N, D), jnp.float32),
    )(partials, partials, W)


def kernel(x, edge_index, edge_values, W):
    ei_flat = edge_index.reshape(2 * E)
    partials = _scatter_add_sc(x, ei_flat, edge_values)
    return _combine_matmul(partials, W)


# speedup vs baseline: 1.0159x; 1.0159x over previous
import functools

import jax
import jax.numpy as jnp
from jax import lax
from jax.experimental import pallas as pl
from jax.experimental.pallas import tpu as pltpu
from jax.experimental.pallas import tpu_sc as plsc

N = 10000
D = 128
E = 320000

NC = 2
NS = 16
NW = NC * NS
LANES = 16

CHUNK = 128
NK = 78
EPW = NK * CHUNK
NTAIL = (E - NW * EPW) // CHUNK
GROUPS = CHUNK // LANES
NBUF = 3

ROWS_PER_TILE = 624
ROWS_TAIL = N - NS * ROWS_PER_TILE

_mesh = plsc.VectorSubcoreMesh(core_axis_name="c", subcore_axis_name="s")


@functools.partial(
    pl.kernel,
    out_type=jax.ShapeDtypeStruct((NC, N, D), jnp.float32),
    mesh=_mesh,
    scratch_types=[
        [pltpu.VMEM((CHUNK,), jnp.int32) for _ in range(NBUF)],
        [pltpu.VMEM((CHUNK,), jnp.int32) for _ in range(NBUF)],
        [pltpu.VMEM((CHUNK,), jnp.float32) for _ in range(NBUF)],
        [pltpu.VMEM((CHUNK, D), jnp.float32) for _ in range(NBUF)],
        pltpu.VMEM_SHARED((N, D), jnp.float32),
        [pltpu.SemaphoreType.DMA for _ in range(NBUF)],
        [pltpu.SemaphoreType.DMA for _ in range(NBUF)],
        [pltpu.SemaphoreType.DMA for _ in range(NBUF)],
        [pltpu.SemaphoreType.DMA for _ in range(NBUF)],
        [pltpu.SemaphoreType.DMA for _ in range(NBUF)],
    ],
)
def _scatter_add_sc(x_hbm, ei_hbm, ev_hbm, out_hbm,
                    srcb, dstb, evb, rows, acc_sh,
                    srcsem, dstsem, evsem, gsem, scsem):
    c = lax.axis_index("c")
    s = lax.axis_index("s")
    wid = s * NC + c
    ebase = wid * EPW

    def zero_body(r, carry):
        for j in range(D // LANES):
            rows[0][r, pl.ds(j * LANES, LANES)] = jnp.zeros(
                (LANES,), jnp.float32)
        return carry

    lax.fori_loop(0, CHUNK, zero_body, 0)
    row0 = s * ROWS_PER_TILE
    for i in range(ROWS_PER_TILE // CHUNK):
        pltpu.sync_copy(rows[0], acc_sh.at[pl.ds(row0 + i * CHUNK, CHUNK)])
    rem = ROWS_PER_TILE % CHUNK
    pltpu.sync_copy(rows[0].at[pl.ds(0, rem)],
                    acc_sh.at[pl.ds(row0 + ROWS_PER_TILE - rem, rem)])

    @pl.when(s == 0)
    def _zero_tail():
        pltpu.sync_copy(rows[0].at[pl.ds(0, ROWS_TAIL)],
                        acc_sh.at[pl.ds(NS * ROWS_PER_TILE, ROWS_TAIL)])

    plsc.subcore_barrier()

    def start_src(z, base):
        pltpu.async_copy(ei_hbm.at[pl.ds(E + base, CHUNK)], srcb[z], srcsem[z])

    def start_dst(z, base):
        pltpu.async_copy(ei_hbm.at[pl.ds(base, CHUNK)], dstb[z], dstsem[z])

    def start_ev(z, base):
        pltpu.async_copy(ev_hbm.at[pl.ds(base, CHUNK)], evb[z], evsem[z])

    def wait_1d(hbm, buf, sem):
        pltpu.make_async_copy(hbm.at[pl.ds(0, CHUNK)], buf, sem).wait()

    def start_gather(z):
        pltpu.async_copy(x_hbm.at[srcb[z]], rows[z], gsem[z])

    def wait_gather(z):
        pltpu.make_async_copy(x_hbm.at[srcb[z]], rows[z], gsem[z]).wait()

    def start_scatter(z):
        pltpu.async_copy(rows[z], acc_sh.at[dstb[z]], scsem[z], add=True)

    def wait_scatter(z):
        pltpu.make_async_copy(rows[z], acc_sh.at[dstb[z]], scsem[z]).wait()

    def scale(rows_b, ev_b):
        def group_body(g, carry):
            ev16 = ev_b[pl.ds(g * LANES, LANES)]
            for i in range(LANES):
                evs = jnp.full((LANES,), ev16[i], jnp.float32)
                e = g * LANES + i
                for j in range(D // LANES):
                    sl = pl.ds(j * LANES, LANES)
                    rows_b[e, sl] = rows_b[e, sl] * evs
            return carry

        lax.fori_loop(0, GROUPS, group_body, 0)

    for kk in range(NBUF):
        start_src(kk, ebase + kk * CHUNK)
        start_ev(kk, ebase + kk * CHUNK)
    for kk in range(2):
        start_dst(kk, ebase + kk * CHUNK)
    for kk in range(2):
        wait_1d(ei_hbm, srcb[kk], srcsem[kk])
        start_gather(kk)

    def ring_body(q, carry):
        for u in range(NBUF):
            k = NBUF * q + u
            z, zp = u, (u - 1) % NBUF
            wait_gather(z)
            wait_1d(ev_hbm, evb[z], evsem[z])
            scale(rows[z], evb[z])
            @pl.when(k > 0)
            def _retire():
                wait_scatter(zp)

            @pl.when(k + 2 < NK)
            def _dst_next():
                start_dst(zp, ebase + (k + 2) * CHUNK)

            @pl.when(k + NBUF < NK)
            def _srcev_next():
                start_src(z, ebase + (k + NBUF) * CHUNK)
                start_ev(z, ebase + (k + NBUF) * CHUNK)

            @pl.when(k + 2 < NK)
            def _gather_next():
                wait_1d(ei_hbm, srcb[zp], srcsem[zp])
                start_gather(zp)

            wait_1d(ei_hbm, dstb[z], dstsem[z])
            start_scatter(z)
        return carry

    lax.fori_loop(0, NK // NBUF, ring_body, 0)
    wait_scatter((NK - 1) % NBUF)

    @pl.when(wid < NTAIL)
    def _tail():
        tbase = NW * EPW + wid * CHUNK
        pltpu.sync_copy(ei_hbm.at[pl.ds(E + tbase, CHUNK)], srcb[0])
        pltpu.sync_copy(ei_hbm.at[pl.ds(tbase, CHUNK)], dstb[0])
        pltpu.sync_copy(ev_hbm.at[pl.ds(tbase, CHUNK)], evb[0])
        pltpu.async_copy(x_hbm.at[srcb[0]], rows[0], gsem[0]).wait()
        scale(rows[0], evb[0])
        pltpu.sync_copy(rows[0], acc_sh.at[dstb[0]], add=True)

    plsc.subcore_barrier()
    pltpu.sync_copy(acc_sh.at[pl.ds(row0, ROWS_PER_TILE)],
                    out_hbm.at[c, pl.ds(row0, ROWS_PER_TILE)])

    @pl.when(s == 0)
    def _drain_tail():
        pltpu.sync_copy(acc_sh.at[pl.ds(NS * ROWS_PER_TILE, ROWS_TAIL)],
                        out_hbm.at[c, pl.ds(NS * ROWS_PER_TILE, ROWS_TAIL)])


BLK = 2000


def _combine_mm_body(p0_ref, p1_ref, w_ref, out_ref):
    a = p0_ref[0] + p1_ref[0]
    out_ref[...] = lax.dot_general(
        a, w_ref[...], (((1,), (1,)), ((), ())),
        preferred_element_type=jnp.float32)


def _combine_matmul(partials, W):
    return pl.pallas_call(
        _combine_mm_body,
        grid=(N // BLK,),
        in_specs=[
            pl.BlockSpec((1, BLK, D), lambda i: (0, i, 0)),
            pl.BlockSpec((1, BLK, D), lambda i: (1, i, 0)),
            pl.BlockSpec((D, D), lambda i: (0, 0)),
        ],
        out_specs=pl.BlockSpec((BLK, D), lambda i: (i, 0)),
        out_shape=jax.ShapeDtypeStruct((N, D), jnp.float32),
    )(partials, partials, W)


def kernel(x, edge_index, edge_values, W):
    ei_flat = edge_index.reshape(2 * E)
    partials = _scatter_add_sc(x, ei_flat, edge_values)
    return _combine_matmul(partials, W)
